# baseline (speedup 1.0000x reference)
import functools

import jax
import jax.numpy as jnp
from jax import lax
from jax.experimental import pallas as pl
from jax.experimental.pallas import tpu as pltpu
from jax.experimental.pallas import tpu_sc as plsc

NC = 2
NS = 16
NW = NC * NS
H = 128
MPAD = 256


def _mm2_body(fb_ref, wi_ref, wh_ref, inp_ref, p_ref):
    inp = jnp.dot(fb_ref[...], wi_ref[...], preferred_element_type=jnp.float32)
    inp_ref[...] = inp
    m = jnp.maximum(inp, 0.0)
    p_ref[...] = jnp.dot(m, wh_ref[...], preferred_element_type=jnp.float32)


def _mm1_body(m_ref, wh_ref, p_ref):
    p_ref[...] = jnp.dot(m_ref[...], wh_ref[...],
                         preferred_element_type=jnp.float32)


def _readout_body(seg_ref, fa_ref, am_ref, woa_ref, wob_ref, bo_ref,
                  out_ref, sums, cnts):
    i = pl.program_id(0)
    nb = pl.num_programs(0)

    @pl.when(i == 0)
    def _():
        sums[...] = jnp.zeros_like(sums)
        cnts[...] = jnp.zeros_like(cnts)

    ah = (jnp.dot(fa_ref[...], woa_ref[...], preferred_element_type=jnp.float32)
          + jnp.dot(am_ref[...], wob_ref[...], preferred_element_type=jnp.float32)
          + bo_ref[...])
    ah = jnp.maximum(ah, 0.0)
    seg = seg_ref[...]
    bn = seg.shape[1]
    mids = lax.broadcasted_iota(jnp.int32, (MPAD, bn), 0)
    onehot = (mids == seg).astype(jnp.float32)
    sums[...] += jnp.dot(onehot, ah, preferred_element_type=jnp.float32)
    cnts[...] += jnp.broadcast_to(
        jnp.sum(onehot, axis=1, keepdims=True), cnts.shape)

    @pl.when(i == nb - 1)
    def _():
        out_ref[...] = sums[...] / jnp.maximum(cnts[...], 1.0)


def _tc_mm2(fb, wi_t, wh_t, be):
    e = fb.shape[0]
    grid = e // be
    return pl.pallas_call(
        _mm2_body,
        grid=(grid,),
        in_specs=[
            pl.BlockSpec((be, H), lambda i: (i, 0)),
            pl.BlockSpec((H, H), lambda i: (0, 0)),
            pl.BlockSpec((H, H), lambda i: (0, 0)),
        ],
        out_specs=[
            pl.BlockSpec((be, H), lambda i: (i, 0)),
            pl.BlockSpec((be, H), lambda i: (i, 0)),
        ],
        out_shape=[
            jax.ShapeDtypeStruct((e, H), jnp.float32),
            jax.ShapeDtypeStruct((e, H), jnp.float32),
        ],
    )(fb, wi_t, wh_t)


def _tc_mm1(m, wh_t, be):
    e = m.shape[0]
    grid = e // be
    return pl.pallas_call(
        _mm1_body,
        grid=(grid,),
        in_specs=[
            pl.BlockSpec((be, H), lambda i: (i, 0)),
            pl.BlockSpec((H, H), lambda i: (0, 0)),
        ],
        out_specs=pl.BlockSpec((be, H), lambda i: (i, 0)),
        out_shape=jax.ShapeDtypeStruct((e, H), jnp.float32),
    )(m, wh_t)


def _tc_readout(seg_pad, fa_pad, am_pad, wo, bo, bn):
    npad = fa_pad.shape[0]
    grid = npad // bn
    woa_t = wo[:, :H].T
    wob_t = wo[:, H:].T
    return pl.pallas_call(
        _readout_body,
        grid=(grid,),
        in_specs=[
            pl.BlockSpec((1, bn), lambda i: (0, i)),
            pl.BlockSpec((bn, H), lambda i: (i, 0)),
            pl.BlockSpec((bn, H), lambda i: (i, 0)),
            pl.BlockSpec((H, H), lambda i: (0, 0)),
            pl.BlockSpec((H, H), lambda i: (0, 0)),
            pl.BlockSpec((1, H), lambda i: (0, 0)),
        ],
        out_specs=pl.BlockSpec((MPAD, H), lambda i: (0, 0)),
        out_shape=jax.ShapeDtypeStruct((MPAD, H), jnp.float32),
        scratch_shapes=[
            pltpu.VMEM((MPAD, H), jnp.float32),
            pltpu.VMEM((MPAD, H), jnp.float32),
        ],
    )(seg_pad, fa_pad, am_pad, woa_t, wob_t, bo.reshape(1, H))


def _nbr_sum_builder(e, npad):
    ch = (npad // NW) // 4
    apw = ch * 4
    mesh = plsc.VectorSubcoreMesh(core_axis_name="c", subcore_axis_name="s")

    @functools.partial(
        pl.kernel,
        mesh=mesh,
        out_type=jax.ShapeDtypeStruct((npad, H), jnp.float32),
        scratch_types=[
            pltpu.VMEM((ch, 128), jnp.int32),
            pltpu.VMEM((128, H), jnp.float32),
            pltpu.VMEM((apw, H), jnp.float32),
            pltpu.SemaphoreType.DMA,
        ],
    )
    def k(table_hbm, idx_hbm, out_hbm, idx_v, gbuf, obuf, sem):
        w = lax.axis_index("s") * NC + lax.axis_index("c")
        pltpu.sync_copy(idx_hbm.at[w], idx_v)

        def chunk(c, carry):
            pltpu.async_copy(table_hbm.at[idx_v.at[c]], gbuf, sem).wait()
            for a in range(4):
                base = a * 32
                accs = [gbuf[base, pl.ds(j * 16, 16)] for j in range(8)]
                for kk in range(1, 32):
                    for j in range(8):
                        accs[j] = accs[j] + gbuf[base + kk, pl.ds(j * 16, 16)]
                row = c * 4 + a
                for j in range(8):
                    obuf[row, pl.ds(j * 16, 16)] = accs[j]
            return carry

        lax.fori_loop(0, ch, chunk, 0)
        pltpu.sync_copy(obuf, out_hbm.at[pl.ds(w * apw, apw)])

    return k


def _msg_builder(e, npad):
    epw = e // NW
    wch = 80
    ch2 = epw // wch
    mesh = plsc.VectorSubcoreMesh(core_axis_name="c", subcore_axis_name="s")

    @functools.partial(
        pl.kernel,
        mesh=mesh,
        out_type=jax.ShapeDtypeStruct((e, H), jnp.float32),
        scratch_types=[
            pltpu.VMEM((ch2, wch), jnp.int32),
            pltpu.VMEM((ch2, wch), jnp.int32),
            pltpu.VMEM((wch, H), jnp.float32),
            pltpu.VMEM((wch, H), jnp.float32),
            pltpu.VMEM((wch, H), jnp.float32),
            pltpu.SemaphoreType.DMA,
            pltpu.SemaphoreType.DMA,
        ],
    )
    def k(a_hbm, p_hbm, inp_hbm, idxa_hbm, idxb_hbm, out_hbm,
          idxa_v, idxb_v, bufi, bufa, bufb, sema, semb):
        w = lax.axis_index("s") * NC + lax.axis_index("c")
        pltpu.sync_copy(idxa_hbm.at[w], idxa_v)
        pltpu.sync_copy(idxb_hbm.at[w], idxb_v)

        def chunk(c, carry):
            e0 = w * epw + c * wch
            ca = pltpu.async_copy(a_hbm.at[idxa_v.at[c]], bufa, sema)
            cb = pltpu.async_copy(p_hbm.at[idxb_v.at[c]], bufb, semb)
            pltpu.sync_copy(inp_hbm.at[pl.ds(e0, wch)], bufi)
            ca.wait()
            cb.wait()

            def row(r, inner):
                for j in range(8):
                    s = pl.ds(j * 16, 16)
                    bufa[r, s] = jnp.maximum(
                        bufi[r, s] + bufa[r, s] - bufb[r, s], 0.0)
                return inner

            lax.fori_loop(0, wch, row, 0, unroll=4)
            pltpu.sync_copy(bufa, out_hbm.at[pl.ds(e0, wch)])
            return carry

        lax.fori_loop(0, ch2, chunk, 0)

    return k


def kernel(f_atoms, f_bonds, a2b, b2a, b2revb, atom_segment_ids,
           W_i, W_h, W_o, b_o):
    n, afdim = f_atoms.shape
    e = f_bonds.shape[0]
    apw = ((n + NW - 1) // NW + 7) // 8 * 8
    npad = NW * apw

    a2b_pad = jnp.pad(a2b.astype(jnp.int32), ((0, npad - n), (0, 0)))
    idx_nbr = a2b_pad.reshape(NW, -1, 128)
    idxa = b2a.astype(jnp.int32).reshape(NW, -1, 80)
    idxb = b2revb.astype(jnp.int32).reshape(NW, -1, 80)

    fa_pad = jnp.pad(f_atoms, ((0, npad - n), (0, 0)))
    seg_pad = jnp.pad(atom_segment_ids.astype(jnp.int32), (0, npad - n),
                      constant_values=MPAD - 1).reshape(1, npad)

    wi_t = W_i.T
    wh_t = W_h.T

    nbr_sum = _nbr_sum_builder(e, npad)
    msg = _msg_builder(e, npad)

    be = 4000
    inp, p1 = _tc_mm2(f_bonds, wi_t, wh_t, be)
    b1 = nbr_sum(p1, idx_nbr)
    m2 = msg(b1, p1, inp, idxa, idxb)
    p2 = _tc_mm1(m2, wh_t, be)
    b2 = nbr_sum(p2, idx_nbr)
    m3 = msg(b2, p2, inp, idxa, idxb)
    am = nbr_sum(m3, idx_nbr)
    out = _tc_readout(seg_pad, fa_pad, am, W_o, b_o, 2048)
    n_mols = 200
    return out[:n_mols]


# speedup vs baseline: 1.2417x; 1.2417x over previous
import functools

import jax
import jax.numpy as jnp
from jax import lax
from jax.experimental import pallas as pl
from jax.experimental.pallas import tpu as pltpu
from jax.experimental.pallas import tpu_sc as plsc

NC = 2
NS = 16
NW = NC * NS
H = 128
MPAD = 256


def _mm2_body(fb_ref, wi_ref, wh_ref, inp_ref, p_ref):
    inp = jnp.dot(fb_ref[...], wi_ref[...], preferred_element_type=jnp.float32)
    inp_ref[...] = inp
    m = jnp.maximum(inp, 0.0)
    p_ref[...] = jnp.dot(m, wh_ref[...], preferred_element_type=jnp.float32)


def _mm1_body(m_ref, wh_ref, p_ref):
    p_ref[...] = jnp.dot(m_ref[...], wh_ref[...],
                         preferred_element_type=jnp.float32)


def _readout_body(seg_ref, fa_ref, am_ref, woa_ref, wob_ref, bo_ref,
                  out_ref, sums, cnts):
    i = pl.program_id(0)
    nb = pl.num_programs(0)

    @pl.when(i == 0)
    def _():
        sums[...] = jnp.zeros_like(sums)
        cnts[...] = jnp.zeros_like(cnts)

    ah = (jnp.dot(fa_ref[...], woa_ref[...], preferred_element_type=jnp.float32)
          + jnp.dot(am_ref[...], wob_ref[...], preferred_element_type=jnp.float32)
          + bo_ref[...])
    ah = jnp.maximum(ah, 0.0)
    seg = seg_ref[...]
    bn = seg.shape[1]
    mids = lax.broadcasted_iota(jnp.int32, (MPAD, bn), 0)
    onehot = (mids == seg).astype(jnp.float32)
    sums[...] += jnp.dot(onehot, ah, preferred_element_type=jnp.float32)
    cnts[...] += jnp.broadcast_to(
        jnp.sum(onehot, axis=1, keepdims=True), cnts.shape)

    @pl.when(i == nb - 1)
    def _():
        out_ref[...] = sums[...] / jnp.maximum(cnts[...], 1.0)


def _tc_mm2(fb, wi_t, wh_t, be):
    e = fb.shape[0]
    grid = e // be
    return pl.pallas_call(
        _mm2_body,
        grid=(grid,),
        in_specs=[
            pl.BlockSpec((be, H), lambda i: (i, 0)),
            pl.BlockSpec((H, H), lambda i: (0, 0)),
            pl.BlockSpec((H, H), lambda i: (0, 0)),
        ],
        out_specs=[
            pl.BlockSpec((be, H), lambda i: (i, 0)),
            pl.BlockSpec((be, H), lambda i: (i, 0)),
        ],
        out_shape=[
            jax.ShapeDtypeStruct((e, H), jnp.float32),
            jax.ShapeDtypeStruct((e, H), jnp.float32),
        ],
    )(fb, wi_t, wh_t)


def _tc_mm1(m, wh_t, be):
    e = m.shape[0]
    grid = e // be
    return pl.pallas_call(
        _mm1_body,
        grid=(grid,),
        in_specs=[
            pl.BlockSpec((be, H), lambda i: (i, 0)),
            pl.BlockSpec((H, H), lambda i: (0, 0)),
        ],
        out_specs=pl.BlockSpec((be, H), lambda i: (i, 0)),
        out_shape=jax.ShapeDtypeStruct((e, H), jnp.float32),
    )(m, wh_t)


def _tc_readout(seg_pad, fa_pad, am_pad, wo, bo, bn):
    npad = fa_pad.shape[0]
    grid = npad // bn
    woa_t = wo[:, :H].T
    wob_t = wo[:, H:].T
    return pl.pallas_call(
        _readout_body,
        grid=(grid,),
        in_specs=[
            pl.BlockSpec((1, bn), lambda i: (0, i)),
            pl.BlockSpec((bn, H), lambda i: (i, 0)),
            pl.BlockSpec((bn, H), lambda i: (i, 0)),
            pl.BlockSpec((H, H), lambda i: (0, 0)),
            pl.BlockSpec((H, H), lambda i: (0, 0)),
            pl.BlockSpec((1, H), lambda i: (0, 0)),
        ],
        out_specs=pl.BlockSpec((MPAD, H), lambda i: (0, 0)),
        out_shape=jax.ShapeDtypeStruct((MPAD, H), jnp.float32),
        scratch_shapes=[
            pltpu.VMEM((MPAD, H), jnp.float32),
            pltpu.VMEM((MPAD, H), jnp.float32),
        ],
    )(seg_pad, fa_pad, am_pad, woa_t, wob_t, bo.reshape(1, H))


def _nbr_sum_builder(e, npad):
    ch = (npad // NW) // 4
    apw = ch * 4
    mesh = plsc.VectorSubcoreMesh(core_axis_name="c", subcore_axis_name="s")

    @functools.partial(
        pl.kernel,
        mesh=mesh,
        out_type=jax.ShapeDtypeStruct((npad, H), jnp.float32),
        scratch_types=[
            pltpu.VMEM((ch, 128), jnp.int32),
            pltpu.VMEM((128, H), jnp.float32),
            pltpu.VMEM((128, H), jnp.float32),
            pltpu.VMEM((apw, H), jnp.float32),
            pltpu.SemaphoreType.DMA,
            pltpu.SemaphoreType.DMA,
        ],
    )
    def k(table_hbm, idx_hbm, out_hbm, idx_v, gbuf0, gbuf1, obuf, sem0, sem1):
        w = lax.axis_index("s") * NC + lax.axis_index("c")
        pltpu.sync_copy(idx_hbm.at[w], idx_v)

        def alu(c, gbuf):
            for a in range(4):
                base = a * 32
                accs = [gbuf[base, pl.ds(j * 16, 16)] for j in range(8)]
                for kk in range(1, 32):
                    for j in range(8):
                        accs[j] = accs[j] + gbuf[base + kk, pl.ds(j * 16, 16)]
                row = c * 4 + a
                for j in range(8):
                    obuf[row, pl.ds(j * 16, 16)] = accs[j]

        pltpu.async_copy(table_hbm.at[idx_v.at[0]], gbuf0, sem0)

        def pair(c2, carry):
            c = 2 * c2
            pltpu.async_copy(table_hbm.at[idx_v.at[c + 1]], gbuf1, sem1)
            pltpu.make_async_copy(table_hbm.at[idx_v.at[c]], gbuf0, sem0).wait()
            alu(c, gbuf0)
            cnext = jnp.minimum(c + 2, ch - 2)
            pltpu.async_copy(table_hbm.at[idx_v.at[cnext]], gbuf0, sem0)
            pltpu.make_async_copy(table_hbm.at[idx_v.at[c + 1]], gbuf1, sem1).wait()
            alu(c + 1, gbuf1)
            return carry

        lax.fori_loop(0, ch // 2, pair, 0)
        pltpu.make_async_copy(table_hbm.at[idx_v.at[ch - 2]], gbuf0, sem0).wait()
        pltpu.sync_copy(obuf, out_hbm.at[pl.ds(w * apw, apw)])

    return k


def _msg_builder(e, npad):
    epw = e // NW
    wch = 80
    ch2 = epw // wch
    mesh = plsc.VectorSubcoreMesh(core_axis_name="c", subcore_axis_name="s")

    @functools.partial(
        pl.kernel,
        mesh=mesh,
        out_type=jax.ShapeDtypeStruct((e, H), jnp.float32),
        scratch_types=[
            pltpu.VMEM((ch2, wch), jnp.int32),
            pltpu.VMEM((ch2, wch), jnp.int32),
            pltpu.VMEM((2, wch, H), jnp.float32),
            pltpu.VMEM((2, wch, H), jnp.float32),
            pltpu.VMEM((2, wch, H), jnp.float32),
            pltpu.SemaphoreType.DMA,
            pltpu.SemaphoreType.DMA,
        ],
    )
    def k(a_hbm, p_hbm, inp_hbm, idxa_hbm, idxb_hbm, out_hbm,
          idxa_v, idxb_v, bufi, bufa, bufb, sem0, sem1):
        w = lax.axis_index("s") * NC + lax.axis_index("c")
        pltpu.sync_copy(idxa_hbm.at[w], idxa_v)
        pltpu.sync_copy(idxb_hbm.at[w], idxb_v)
        sems = [sem0, sem1]

        def issue(c, slot):
            e0 = w * epw + c * wch
            pltpu.async_copy(a_hbm.at[idxa_v.at[c]], bufa.at[slot], sems[slot])
            pltpu.async_copy(p_hbm.at[idxb_v.at[c]], bufb.at[slot], sems[slot])
            pltpu.async_copy(inp_hbm.at[pl.ds(e0, wch)], bufi.at[slot],
                             sems[slot])

        def drain(c, slot):
            pltpu.make_async_copy(a_hbm.at[idxa_v.at[c]], bufa.at[slot],
                                  sems[slot]).wait()
            pltpu.make_async_copy(p_hbm.at[idxb_v.at[c]], bufb.at[slot],
                                  sems[slot]).wait()
            e0 = w * epw + c * wch
            pltpu.make_async_copy(inp_hbm.at[pl.ds(e0, wch)], bufi.at[slot],
                                  sems[slot]).wait()

        def alu_store(c, slot):
            def row(r, inner):
                for j in range(8):
                    s = pl.ds(j * 16, 16)
                    bufa[slot, r, s] = jnp.maximum(
                        bufi[slot, r, s] + bufa[slot, r, s] - bufb[slot, r, s],
                        0.0)
                return inner

            lax.fori_loop(0, wch, row, 0, unroll=4)
            e0 = w * epw + c * wch
            pltpu.sync_copy(bufa.at[slot], out_hbm.at[pl.ds(e0, wch)])

        issue(0, 0)

        def pair(c2, carry):
            c = 2 * c2
            issue(c + 1, 1)
            drain(c, 0)
            alu_store(c, 0)
            issue(c + 2, 0)
            drain(c + 1, 1)
            alu_store(c + 1, 1)
            return carry

        lax.fori_loop(0, (ch2 - 1) // 2, pair, 0)
        drain(ch2 - 1, 0)
        alu_store(ch2 - 1, 0)

    return k


def kernel(f_atoms, f_bonds, a2b, b2a, b2revb, atom_segment_ids,
           W_i, W_h, W_o, b_o):
    n, afdim = f_atoms.shape
    e = f_bonds.shape[0]
    apw = ((n + NW - 1) // NW + 7) // 8 * 8
    npad = NW * apw

    a2b_pad = jnp.pad(a2b.astype(jnp.int32), ((0, npad - n), (0, 0)))
    idx_nbr = a2b_pad.reshape(NW, -1, 128)
    idxa = b2a.astype(jnp.int32).reshape(NW, -1, 80)
    idxb = b2revb.astype(jnp.int32).reshape(NW, -1, 80)

    fa_pad = jnp.pad(f_atoms, ((0, npad - n), (0, 0)))
    seg_pad = jnp.pad(atom_segment_ids.astype(jnp.int32), (0, npad - n),
                      constant_values=MPAD - 1).reshape(1, npad)

    wi_t = W_i.T
    wh_t = W_h.T

    nbr_sum = _nbr_sum_builder(e, npad)
    msg = _msg_builder(e, npad)

    be = 4000
    inp, p1 = _tc_mm2(f_bonds, wi_t, wh_t, be)
    b1 = nbr_sum(p1, idx_nbr)
    m2 = msg(b1, p1, inp, idxa, idxb)
    p2 = _tc_mm1(m2, wh_t, be)
    b2 = nbr_sum(p2, idx_nbr)
    m3 = msg(b2, p2, inp, idxa, idxb)
    am = nbr_sum(m3, idx_nbr)
    out = _tc_readout(seg_pad, fa_pad, am, W_o, b_o, 2048)
    n_mols = 200
    return out[:n_mols]
